# baseline (speedup 1.0000x reference)
import jax
import jax.numpy as jnp
from jax import lax
from jax.experimental import pallas as pl
from jax.experimental.pallas import tpu as pltpu
from jax.experimental.pallas import tpu_sc as plsc

_N = 10000
_E = 160000
_D = 256
_BLK = 400

_NC = 2
_NS = 16
_L = 16
_EPAD = 160256
_EPT = _EPAD // _NS
_ACH = 512
_NACH = _EPT // _ACH
_ATL = _EPT - _NACH * _ACH
_PCAP = 10240
_PBCAP = _PCAP + _ACH
_PACK = 16384
_SENT = 16383 * _PACK
_OWN = 320
_PLANE = _OWN * 2 * _NS
_GB = 32
_PKCAP = _ACH + _GB


def _seg_body(x_hbm, src_hbm, dst_hbm, glp_hbm, z_hbm, s_out, cnt_out,
              glp_v, pubbuf, ebuf, pk_l, gsta0, gsta1, lsta0,
              lsta1, lbuf, lens_l, cnt_l, rows0, rows1, acc, sem0, sem1,
              pub_s, lens_s):
    srcb = ebuf
    dstb = pk_l.at[pl.ds(0, _ACH)]
    cid = lax.axis_index("c")
    sid = lax.axis_index("s")
    iota = lax.broadcasted_iota(jnp.int32, (_L,), 0)
    ones_f = jnp.ones((_L,), jnp.float32)
    zero_i = jnp.zeros((_L,), jnp.int32)

    pltpu.sync_copy(glp_hbm, glp_v)

    estart = sid * _EPT

    def stepf(st, ptrv):
        s = srcb[pl.ds(st * _L, _L)]
        d = dstb[pl.ds(st * _L, _L)]
        w = plsc.load_gather(glp_v, [s >> 2])
        gv = (w >> ((s & 3) * 8)) & 255
        m = (gv == cid) & (d < _N)
        pv = d * _PACK + s
        mi = m.astype(jnp.int32)
        pos = ptrv + plsc.cumsum(mi) - mi
        plsc.store_scatter(pubbuf, [pos], pv, mask=m)
        return ptrv + plsc.all_reduce_population_count(m)

    def chunk(ch, ptrv):
        off = estart + ch * _ACH
        pltpu.sync_copy(src_hbm.at[pl.ds(off, _ACH)], srcb)
        pltpu.sync_copy(dst_hbm.at[pl.ds(off, _ACH)], dstb)
        return lax.fori_loop(0, _ACH // _L, stepf, ptrv)

    with jax.named_scope("phaseA"):
        ptrv = lax.fori_loop(0, _NACH, chunk, jnp.zeros((_L,), jnp.int32))
    off = estart + _NACH * _ACH
    pltpu.sync_copy(src_hbm.at[pl.ds(off, _ATL)], srcb.at[pl.ds(0, _ATL)])
    pltpu.sync_copy(dst_hbm.at[pl.ds(off, _ATL)], dstb.at[pl.ds(0, _ATL)])
    ptrv = lax.fori_loop(0, _ATL // _L, stepf, ptrv)
    m_end = jnp.sum(jnp.where(iota == 0, ptrv, 0))

    sent = jnp.full((_L,), _SENT, jnp.int32)
    for k in range(_ACH // _L):
        pubbuf[pl.ds(m_end + k * _L, _L)] = sent
    npub = (m_end + _ACH - 1) // _ACH

    def pub(ch, _):
        pltpu.sync_copy(pubbuf.at[pl.ds(ch * _ACH, _ACH)],
                        pub_s.at[pl.ds(sid * _PCAP + ch * _ACH, _ACH)])
        return 0

    lax.fori_loop(0, npub, pub, 0)
    lbuf[pl.ds(0, _L)] = zero_i + npub * _ACH
    pltpu.sync_copy(lbuf, lens_s.at[pl.ds(sid * _L, _L)])
    plsc.subcore_barrier()

    pltpu.sync_copy(lens_s, lens_l)

    def start(b, gsta_b, rows_b, sem_b):
        for k in range(_GB // _L):
            pv = pk_l[pl.ds(b * _GB + k * _L, _L)]
            gsta_b[0, pl.ds(k * _L, _L)] = pv & (_PACK - 1)
        pltpu.async_copy(x_hbm.at[gsta_b.at[0]], rows_b, sem_b)

    def snap_rows(b, lsta_b):
        for k in range(_GB // _L):
            lsta_b[0, pl.ds(k * _L, _L)] = pk_l[pl.ds(b * _GB + k * _L,
                                                      _L)] >> 14

    def wait(rows_b, sem_b):
        pltpu.make_async_copy(x_hbm.at[pl.ds(0, _GB)], rows_b, sem_b).wait()

    def accum(rows_b, lsta_b, nrows):
        def prow(i, _):
            zi16 = jnp.zeros((_L,), jnp.int32)
            rspl = plsc.load_gather(lsta_b, [zi16, zi16 + i])
            rbase = rspl * _D + iota
            for c in range(_D // _L):
                v = rows_b[i, pl.ds(c * _L, _L)]
                plsc.addupdate_scatter(acc, [rbase + c * _L], v)
            return 0

        lax.fori_loop(0, nrows, prow, 0)

    def flush_many(nfb):
        @pl.when(nfb > 0)
        def _():
            start(0, gsta0, rows0, sem0)

        def pair(pp, _):
            b0 = pp * 2

            @pl.when(b0 + 1 < nfb)
            def _():
                start(b0 + 1, gsta1, rows1, sem1)

            snap_rows(b0, lsta0)
            wait(rows0, sem0)
            accum(rows0, lsta0, _GB)

            @pl.when(b0 + 1 < nfb)
            def _():
                @pl.when(b0 + 2 < nfb)
                def _():
                    start(b0 + 2, gsta0, rows0, sem0)

                snap_rows(b0 + 1, lsta1)
                wait(rows1, sem1)
                accum(rows1, lsta1, _GB)

            return 0

        lax.fori_loop(0, (nfb + 1) // 2, pair, 0)

    for p in range(2):
        r0 = (p * _NS + sid) * _OWN
        with jax.named_scope("zero"):
            for k in range(_OWN * _D // 4096):
                pltpu.sync_copy(z_hbm, acc.at[pl.ds(k * 4096, 4096)])
        for k in range(_OWN // _L):
            cnt_l[pl.ds(k * _L, _L)] = jnp.zeros((_L,), jnp.float32)

        def publisher(j, ptr2, r0=r0):
            mv = lens_l[pl.ds(j * _L, _L)]
            m_j = jnp.sum(jnp.where(iota == 0, mv, 0))

            def chunk2(ch, ptr2, j=j, r0=r0):
                pltpu.sync_copy(
                    pub_s.at[pl.ds(j * _PCAP + ch * _ACH, _ACH)], ebuf)

                def step2(st, ptr2v, r0=r0):
                    v = ebuf[pl.ds(st * _L, _L)]
                    d = v >> 14
                    s = v & (_PACK - 1)
                    ld = d - r0
                    own = (ld >= 0) & (ld < _OWN)
                    lv = ld * _PACK + s
                    oi = own.astype(jnp.int32)
                    pos = ptr2v + plsc.cumsum(oi) - oi
                    plsc.store_scatter(pk_l, [pos], lv, mask=own)
                    plsc.addupdate_scatter(cnt_l, [ld], ones_f, mask=own)
                    return ptr2v + plsc.all_reduce_population_count(own)

                with jax.named_scope("scan"):
                    ptr2v = lax.fori_loop(0, _ACH // _L, step2,
                                          jnp.zeros((_L,), jnp.int32) + ptr2)
                    ptr2 = jnp.sum(jnp.where(iota == 0, ptr2v, 0))
                nfb = ptr2 // _GB
                with jax.named_scope("flush"):
                    flush_many(nfb)
                for k in range(_GB // _L):
                    t = pk_l[pl.ds(nfb * _GB + k * _L, _L)]
                    pk_l[pl.ds(k * _L, _L)] = t
                return ptr2 & (_GB - 1)

            return lax.fori_loop(0, m_j // _ACH, chunk2, ptr2)

        with jax.named_scope("phaseB"):
            rem = lax.fori_loop(0, _NS, publisher, jnp.int32(0))
        for k in range(_GB // _L):
            pk_l[pl.ds(rem + k * _L, _L)] = zero_i
        start(0, gsta0, rows0, sem0)
        snap_rows(0, lsta0)
        wait(rows0, sem0)
        accum(rows0, lsta0, rem)

        pltpu.sync_copy(
            acc, s_out.at[pl.ds((cid * _PLANE + r0) * _D, _OWN * _D)])
        pltpu.sync_copy(
            cnt_l, cnt_out.at[pl.ds(cid * _PLANE + r0, _OWN)])


def _make_seg():
    mesh = plsc.VectorSubcoreMesh(core_axis_name="c", subcore_axis_name="s",
                                  num_cores=_NC, num_subcores=_NS)
    scratch = [
        pltpu.VMEM((_N // 4,), jnp.int32),
        pltpu.VMEM((_PBCAP,), jnp.int32),
        pltpu.VMEM((_ACH,), jnp.int32),
        pltpu.VMEM((_PKCAP,), jnp.int32),
        pltpu.VMEM((1, _GB), jnp.int32),
        pltpu.VMEM((1, _GB), jnp.int32),
        pltpu.VMEM((1, _GB), jnp.int32),
        pltpu.VMEM((1, _GB), jnp.int32),
        pltpu.VMEM((_L,), jnp.int32),
        pltpu.VMEM((_NS * _L,), jnp.int32),
        pltpu.VMEM((_OWN,), jnp.float32),
        pltpu.VMEM((_GB, _D), jnp.float32),
        pltpu.VMEM((_GB, _D), jnp.float32),
        pltpu.VMEM((_OWN * _D,), jnp.float32),
        pltpu.SemaphoreType.DMA,
        pltpu.SemaphoreType.DMA,
        pltpu.VMEM_SHARED((_NS * _PCAP,), jnp.int32),
        pltpu.VMEM_SHARED((_NS * _L,), jnp.int32),
    ]
    return pl.kernel(
        _seg_body,
        out_type=(jax.ShapeDtypeStruct((_NC * _PLANE * _D,), jnp.float32),
                  jax.ShapeDtypeStruct((_NC * _PLANE,), jnp.float32)),
        mesh=mesh,
        scratch_types=scratch,
        compiler_params=pltpu.CompilerParams(needs_layout_passes=False),
        name="seg_sum")


_seg = _make_seg()


def _layer_body(x_ref, s0_ref, s1_ref, c_ref, ws_ref, w0_ref, w1_ref, b_ref,
                o_ref):
    inv = 1.0 / jnp.maximum(c_ref[...], 1.0)
    acc = jnp.dot(x_ref[...], ws_ref[...], preferred_element_type=jnp.float32)
    acc += jnp.dot(inv[:, 0:1] * s0_ref[0], w0_ref[...],
                   preferred_element_type=jnp.float32)
    acc += jnp.dot(inv[:, 1:2] * s1_ref[0], w1_ref[...],
                   preferred_element_type=jnp.float32)
    o_ref[...] = jnp.maximum(acc + b_ref[...], 0.0)


def _final_body(x_ref, s0_ref, s1_ref, c_ref, ws_ref, w0_ref, w1_ref, b_ref,
                wc_ref, bc_ref, o_ref):
    inv = 1.0 / jnp.maximum(c_ref[...], 1.0)
    acc = jnp.dot(x_ref[...], ws_ref[...], preferred_element_type=jnp.float32)
    acc += jnp.dot(inv[:, 0:1] * s0_ref[0], w0_ref[...],
                   preferred_element_type=jnp.float32)
    acc += jnp.dot(inv[:, 1:2] * s1_ref[0], w1_ref[...],
                   preferred_element_type=jnp.float32)
    x2 = jnp.maximum(acc + b_ref[...], 0.0)
    o_ref[...] = jnp.dot(x2, wc_ref[...],
                         preferred_element_type=jnp.float32) + bc_ref[...]


def _dense_layer(x, S, cnt, w_self, w_grp, b, wc=None, bc=None):
    grid = _N // _BLK
    in_specs = [
        pl.BlockSpec((_BLK, _D), lambda i: (i, 0)),
        pl.BlockSpec((1, _BLK, _D), lambda i: (0, i, 0)),
        pl.BlockSpec((1, _BLK, _D), lambda i: (1, i, 0)),
        pl.BlockSpec((_BLK, 2), lambda i: (i, 0)),
        pl.BlockSpec((_D, _D), lambda i: (0, 0)),
        pl.BlockSpec((_D, _D), lambda i: (0, 0)),
        pl.BlockSpec((_D, _D), lambda i: (0, 0)),
        pl.BlockSpec((1, _D), lambda i: (0, 0)),
    ]
    args = [x, S, S, cnt, w_self, w_grp[0], w_grp[1], b.reshape(1, _D)]
    if wc is None:
        body, out_d = _layer_body, _D
    else:
        body, out_d = _final_body, wc.shape[1]
        in_specs += [
            pl.BlockSpec((_D, out_d), lambda i: (0, 0)),
            pl.BlockSpec((1, out_d), lambda i: (0, 0)),
        ]
        args += [wc, bc.reshape(1, out_d)]
    return pl.pallas_call(
        body,
        grid=(grid,),
        in_specs=in_specs,
        out_specs=pl.BlockSpec((_BLK, out_d), lambda i: (i, 0)),
        out_shape=jax.ShapeDtypeStruct((_N, out_d), jnp.float32),
    )(*args)


def kernel(h, edge_index, group_labels, W_grp1, W_self1, b1,
           W_grp2, W_self2, b2, Wc, bc):
    npad = _EPAD - _E
    src = jnp.concatenate([edge_index[0], jnp.zeros((npad,), jnp.int32)])
    dst = jnp.concatenate([edge_index[1], jnp.full((npad,), _N, jnp.int32)])
    glp = (group_labels.reshape(_N // 4, 4)
           << jnp.array([0, 8, 16, 24], jnp.int32)).sum(
               axis=1, dtype=jnp.int32)
    zf = jnp.zeros((4096,), jnp.float32)

    Sf, cf = _seg(h, src, dst, glp, zf)
    S1 = Sf.reshape(_NC, _PLANE, _D)
    cnt_pl = cf.reshape(_NC, _PLANE)
    cnt = jnp.stack([cnt_pl[0, :_N], cnt_pl[1, :_N]], axis=1)

    x1 = _dense_layer(h, S1, cnt, W_self1, W_grp1, b1)
    Sf2, _ = _seg(x1, src, dst, glp, zf)
    S2 = Sf2.reshape(_NC, _PLANE, _D)
    return _dense_layer(x1, S2, cnt, W_self2, W_grp2, b2, Wc, bc)


# speedup vs baseline: 1.2114x; 1.2114x over previous
import jax
import jax.numpy as jnp
from jax import lax
from jax.experimental import pallas as pl
from jax.experimental.pallas import tpu as pltpu
from jax.experimental.pallas import tpu_sc as plsc

_N = 10000
_E = 160000
_D = 256
_BLK = 400

_NC = 2
_NS = 16
_L = 16
_EPAD = 160256
_EPT = _EPAD // _NS
_ACH = 512
_NACH = _EPT // _ACH
_ATL = _EPT - _NACH * _ACH
_PCAP = 10240
_PBCAP = _PCAP + _ACH
_PACK = 16384
_SENT = 16383 * _PACK
_OWN = 320
_PLANE = _OWN * 2 * _NS
_GB = 32
_FLUSH_AT = 4096
_PKCAP = _FLUSH_AT + _ACH + _GB


def _seg_body(x_hbm, src_hbm, dst_hbm, glp_hbm, z_hbm, s_out, cnt_out,
              glp_v, pubbuf, ebuf, pk_l, gsta0, gsta1, lsta0,
              lsta1, lbuf, lens_l, cnt_l, rows0, rows1, acc, sem0, sem1,
              pub_s, lens_s):
    srcb = ebuf
    dstb = pk_l.at[pl.ds(0, _ACH)]
    cid = lax.axis_index("c")
    sid = lax.axis_index("s")
    iota = lax.broadcasted_iota(jnp.int32, (_L,), 0)
    ones_f = jnp.ones((_L,), jnp.float32)
    zero_i = jnp.zeros((_L,), jnp.int32)

    pltpu.sync_copy(glp_hbm, glp_v)

    estart = sid * _EPT

    def stepf(st, ptrv):
        s = srcb[pl.ds(st * _L, _L)]
        d = dstb[pl.ds(st * _L, _L)]
        w = plsc.load_gather(glp_v, [s >> 2])
        gv = (w >> ((s & 3) * 8)) & 255
        m = (gv == cid) & (d < _N)
        pv = d * _PACK + s
        mi = m.astype(jnp.int32)
        pos = ptrv + plsc.cumsum(mi) - mi
        plsc.store_scatter(pubbuf, [pos], pv, mask=m)
        return ptrv + plsc.all_reduce_population_count(m)

    def chunk(ch, ptrv):
        off = estart + ch * _ACH
        pltpu.sync_copy(src_hbm.at[pl.ds(off, _ACH)], srcb)
        pltpu.sync_copy(dst_hbm.at[pl.ds(off, _ACH)], dstb)
        return lax.fori_loop(0, _ACH // _L, stepf, ptrv)

    with jax.named_scope("phaseA"):
        ptrv = lax.fori_loop(0, _NACH, chunk, jnp.zeros((_L,), jnp.int32))
    off = estart + _NACH * _ACH
    pltpu.sync_copy(src_hbm.at[pl.ds(off, _ATL)], srcb.at[pl.ds(0, _ATL)])
    pltpu.sync_copy(dst_hbm.at[pl.ds(off, _ATL)], dstb.at[pl.ds(0, _ATL)])
    ptrv = lax.fori_loop(0, _ATL // _L, stepf, ptrv)
    m_end = jnp.sum(jnp.where(iota == 0, ptrv, 0))

    sent = jnp.full((_L,), _SENT, jnp.int32)
    for k in range(_ACH // _L):
        pubbuf[pl.ds(m_end + k * _L, _L)] = sent
    npub = (m_end + _ACH - 1) // _ACH

    def pub(ch, _):
        pltpu.sync_copy(pubbuf.at[pl.ds(ch * _ACH, _ACH)],
                        pub_s.at[pl.ds(sid * _PCAP + ch * _ACH, _ACH)])
        return 0

    lax.fori_loop(0, npub, pub, 0)
    lbuf[pl.ds(0, _L)] = zero_i + npub * _ACH
    pltpu.sync_copy(lbuf, lens_s.at[pl.ds(sid * _L, _L)])
    plsc.subcore_barrier()

    pltpu.sync_copy(lens_s, lens_l)

    def start(b, gsta_b, rows_b, sem_b):
        for k in range(_GB // _L):
            pv = pk_l[pl.ds(b * _GB + k * _L, _L)]
            gsta_b[0, pl.ds(k * _L, _L)] = pv & (_PACK - 1)
        pltpu.async_copy(x_hbm.at[gsta_b.at[0]], rows_b, sem_b)

    def snap_rows(b, lsta_b):
        for k in range(_GB // _L):
            lsta_b[0, pl.ds(k * _L, _L)] = pk_l[pl.ds(b * _GB + k * _L,
                                                      _L)] >> 14

    def wait(rows_b, sem_b):
        pltpu.make_async_copy(x_hbm.at[pl.ds(0, _GB)], rows_b, sem_b).wait()

    def accum(rows_b, lsta_b, nrows):
        def prow(i, _):
            zi16 = jnp.zeros((_L,), jnp.int32)
            rspl = plsc.load_gather(lsta_b, [zi16, zi16 + i])
            rbase = rspl * _D + iota
            for c in range(_D // _L):
                v = rows_b[i, pl.ds(c * _L, _L)]
                plsc.addupdate_scatter(acc, [rbase + c * _L], v)
            return 0

        lax.fori_loop(0, nrows, prow, 0)

    def flush_many(nfb):
        @pl.when(nfb > 0)
        def _():
            start(0, gsta0, rows0, sem0)

        def pair(pp, _):
            b0 = pp * 2

            @pl.when(b0 + 1 < nfb)
            def _():
                start(b0 + 1, gsta1, rows1, sem1)

            snap_rows(b0, lsta0)
            wait(rows0, sem0)
            accum(rows0, lsta0, _GB)

            @pl.when(b0 + 1 < nfb)
            def _():
                @pl.when(b0 + 2 < nfb)
                def _():
                    start(b0 + 2, gsta0, rows0, sem0)

                snap_rows(b0 + 1, lsta1)
                wait(rows1, sem1)
                accum(rows1, lsta1, _GB)

            return 0

        lax.fori_loop(0, (nfb + 1) // 2, pair, 0)

    for p in range(2):
        r0 = (p * _NS + sid) * _OWN
        with jax.named_scope("zero"):
            for k in range(_OWN * _D // 4096):
                pltpu.sync_copy(z_hbm, acc.at[pl.ds(k * 4096, 4096)])
        for k in range(_OWN // _L):
            cnt_l[pl.ds(k * _L, _L)] = jnp.zeros((_L,), jnp.float32)

        def publisher(j, ptr2, r0=r0):
            mv = lens_l[pl.ds(j * _L, _L)]
            m_j = jnp.sum(jnp.where(iota == 0, mv, 0))

            def chunk2(ch, ptr2, j=j, r0=r0):
                pltpu.sync_copy(
                    pub_s.at[pl.ds(j * _PCAP + ch * _ACH, _ACH)], ebuf)

                def step2(st, ptr2v, r0=r0):
                    v = ebuf[pl.ds(st * _L, _L)]
                    d = v >> 14
                    s = v & (_PACK - 1)
                    ld = d - r0
                    own = (ld >= 0) & (ld < _OWN)
                    lv = ld * _PACK + s
                    oi = own.astype(jnp.int32)
                    pos = ptr2v + plsc.cumsum(oi) - oi
                    plsc.store_scatter(pk_l, [pos], lv, mask=own)
                    plsc.addupdate_scatter(cnt_l, [ld], ones_f, mask=own)
                    return ptr2v + plsc.all_reduce_population_count(own)

                with jax.named_scope("scan"):
                    ptr2v = lax.fori_loop(0, _ACH // _L, step2,
                                          jnp.zeros((_L,), jnp.int32) + ptr2)
                    ptr2 = jnp.sum(jnp.where(iota == 0, ptr2v, 0))
                def do_flush(ptr2=ptr2):
                    nfb = ptr2 // _GB
                    with jax.named_scope("flush"):
                        flush_many(nfb)
                    for k in range(_GB // _L):
                        t = pk_l[pl.ds(nfb * _GB + k * _L, _L)]
                        pk_l[pl.ds(k * _L, _L)] = t

                flush_now = ptr2 >= _FLUSH_AT
                pl.when(flush_now)(do_flush)
                return jnp.where(flush_now, ptr2 & (_GB - 1), ptr2)

            return lax.fori_loop(0, m_j // _ACH, chunk2, ptr2)

        with jax.named_scope("phaseB"):
            rem = lax.fori_loop(0, _NS, publisher, jnp.int32(0))
        nfbf = rem // _GB
        flush_many(nfbf)
        for k in range(_GB // _L):
            t = pk_l[pl.ds(nfbf * _GB + k * _L, _L)]
            pk_l[pl.ds(k * _L, _L)] = t
        remp = rem & (_GB - 1)
        for k in range(_GB // _L):
            pk_l[pl.ds(remp + k * _L, _L)] = zero_i
        start(0, gsta0, rows0, sem0)
        snap_rows(0, lsta0)
        wait(rows0, sem0)
        accum(rows0, lsta0, remp)

        pltpu.sync_copy(
            acc, s_out.at[pl.ds((cid * _PLANE + r0) * _D, _OWN * _D)])
        pltpu.sync_copy(
            cnt_l, cnt_out.at[pl.ds(cid * _PLANE + r0, _OWN)])


def _make_seg():
    mesh = plsc.VectorSubcoreMesh(core_axis_name="c", subcore_axis_name="s",
                                  num_cores=_NC, num_subcores=_NS)
    scratch = [
        pltpu.VMEM((_N // 4,), jnp.int32),
        pltpu.VMEM((_PBCAP,), jnp.int32),
        pltpu.VMEM((_ACH,), jnp.int32),
        pltpu.VMEM((_PKCAP,), jnp.int32),
        pltpu.VMEM((1, _GB), jnp.int32),
        pltpu.VMEM((1, _GB), jnp.int32),
        pltpu.VMEM((1, _GB), jnp.int32),
        pltpu.VMEM((1, _GB), jnp.int32),
        pltpu.VMEM((_L,), jnp.int32),
        pltpu.VMEM((_NS * _L,), jnp.int32),
        pltpu.VMEM((_OWN,), jnp.float32),
        pltpu.VMEM((_GB, _D), jnp.float32),
        pltpu.VMEM((_GB, _D), jnp.float32),
        pltpu.VMEM((_OWN * _D,), jnp.float32),
        pltpu.SemaphoreType.DMA,
        pltpu.SemaphoreType.DMA,
        pltpu.VMEM_SHARED((_NS * _PCAP,), jnp.int32),
        pltpu.VMEM_SHARED((_NS * _L,), jnp.int32),
    ]
    return pl.kernel(
        _seg_body,
        out_type=(jax.ShapeDtypeStruct((_NC * _PLANE * _D,), jnp.float32),
                  jax.ShapeDtypeStruct((_NC * _PLANE,), jnp.float32)),
        mesh=mesh,
        scratch_types=scratch,
        compiler_params=pltpu.CompilerParams(needs_layout_passes=False),
        name="seg_sum")


_seg = _make_seg()


def _layer_body(x_ref, s0_ref, s1_ref, c_ref, ws_ref, w0_ref, w1_ref, b_ref,
                o_ref):
    inv = 1.0 / jnp.maximum(c_ref[...], 1.0)
    acc = jnp.dot(x_ref[...], ws_ref[...], preferred_element_type=jnp.float32)
    acc += jnp.dot(inv[:, 0:1] * s0_ref[0], w0_ref[...],
                   preferred_element_type=jnp.float32)
    acc += jnp.dot(inv[:, 1:2] * s1_ref[0], w1_ref[...],
                   preferred_element_type=jnp.float32)
    o_ref[...] = jnp.maximum(acc + b_ref[...], 0.0)


def _final_body(x_ref, s0_ref, s1_ref, c_ref, ws_ref, w0_ref, w1_ref, b_ref,
                wc_ref, bc_ref, o_ref):
    inv = 1.0 / jnp.maximum(c_ref[...], 1.0)
    acc = jnp.dot(x_ref[...], ws_ref[...], preferred_element_type=jnp.float32)
    acc += jnp.dot(inv[:, 0:1] * s0_ref[0], w0_ref[...],
                   preferred_element_type=jnp.float32)
    acc += jnp.dot(inv[:, 1:2] * s1_ref[0], w1_ref[...],
                   preferred_element_type=jnp.float32)
    x2 = jnp.maximum(acc + b_ref[...], 0.0)
    o_ref[...] = jnp.dot(x2, wc_ref[...],
                         preferred_element_type=jnp.float32) + bc_ref[...]


def _dense_layer(x, S, cnt, w_self, w_grp, b, wc=None, bc=None):
    grid = _N // _BLK
    in_specs = [
        pl.BlockSpec((_BLK, _D), lambda i: (i, 0)),
        pl.BlockSpec((1, _BLK, _D), lambda i: (0, i, 0)),
        pl.BlockSpec((1, _BLK, _D), lambda i: (1, i, 0)),
        pl.BlockSpec((_BLK, 2), lambda i: (i, 0)),
        pl.BlockSpec((_D, _D), lambda i: (0, 0)),
        pl.BlockSpec((_D, _D), lambda i: (0, 0)),
        pl.BlockSpec((_D, _D), lambda i: (0, 0)),
        pl.BlockSpec((1, _D), lambda i: (0, 0)),
    ]
    args = [x, S, S, cnt, w_self, w_grp[0], w_grp[1], b.reshape(1, _D)]
    if wc is None:
        body, out_d = _layer_body, _D
    else:
        body, out_d = _final_body, wc.shape[1]
        in_specs += [
            pl.BlockSpec((_D, out_d), lambda i: (0, 0)),
            pl.BlockSpec((1, out_d), lambda i: (0, 0)),
        ]
        args += [wc, bc.reshape(1, out_d)]
    return pl.pallas_call(
        body,
        grid=(grid,),
        in_specs=in_specs,
        out_specs=pl.BlockSpec((_BLK, out_d), lambda i: (i, 0)),
        out_shape=jax.ShapeDtypeStruct((_N, out_d), jnp.float32),
    )(*args)


def kernel(h, edge_index, group_labels, W_grp1, W_self1, b1,
           W_grp2, W_self2, b2, Wc, bc):
    npad = _EPAD - _E
    src = jnp.concatenate([edge_index[0], jnp.zeros((npad,), jnp.int32)])
    dst = jnp.concatenate([edge_index[1], jnp.full((npad,), _N, jnp.int32)])
    glp = (group_labels.reshape(_N // 4, 4)
           << jnp.array([0, 8, 16, 24], jnp.int32)).sum(
               axis=1, dtype=jnp.int32)
    zf = jnp.zeros((4096,), jnp.float32)

    Sf, cf = _seg(h, src, dst, glp, zf)
    S1 = Sf.reshape(_NC, _PLANE, _D)
    cnt_pl = cf.reshape(_NC, _PLANE)
    cnt = jnp.stack([cnt_pl[0, :_N], cnt_pl[1, :_N]], axis=1)

    x1 = _dense_layer(h, S1, cnt, W_self1, W_grp1, b1)
    Sf2, _ = _seg(x1, src, dst, glp, zf)
    S2 = Sf2.reshape(_NC, _PLANE, _D)
    return _dense_layer(x1, S2, cnt, W_self2, W_grp2, b2, Wc, bc)
